# baseline (speedup 1.0000x reference)
import functools

import jax
import jax.numpy as jnp
from jax import lax
from jax.experimental import pallas as pl
from jax.experimental.pallas import tpu as pltpu
from jax.experimental.pallas import tpu_sc as plsc

B = 16384
DIM = 64
NUM_ROWS = 1000000
BAND = 128
NFULL = NUM_ROWS // BAND
EDGE0 = NFULL * BAND
NBANDS = NFULL + 1
NUM_CORES = 2
NUM_SUBCORES = 16
NW = NUM_CORES * NUM_SUBCORES
RPW = B // NW
LANES = 16
SUPER = 4
NSUPER = 62
GROWS = B + 16
SENT = B
PW = 128


def _scan_pass(idx_hbm, table_hbm, edge_hbm, g_hbm, xbuf, hl, rb, arena,
               fsem, ssem, wid, lo_w, nb_w):
    lanes = lax.iota(jnp.int32, LANES)
    c127 = jnp.full((LANES,), 127, jnp.int32)
    lo_v = jnp.full((LANES,), 1, jnp.int32) * lo_w
    nb_v = jnp.full((LANES,), 1, jnp.int32) * nb_w

    pltpu.sync_copy(idx_hbm, xbuf)

    def scan_chunk(c, cnt):
        v = xbuf[pl.ds(c * LANES, LANES)]
        pos = lanes + c * LANES
        bl = lax.shift_right_logical(v, 7) - lo_v
        m = (bl >= 0) & (bl < nb_v)
        h = (lax.shift_left(bl, 22) |
             lax.shift_left(v & c127, 15) | pos)
        plsc.store_compressed(hl.at[pl.ds(cnt, LANES)], h, mask=m)
        nhit = plsc.all_reduce_population_count(m)
        return cnt + nhit[0]

    cnt = lax.fori_loop(0, B // LANES, scan_chunk, jnp.int32(0))
    hl[pl.ds(cnt, LANES)] = jnp.full((LANES,), SENT, jnp.int32)
    nch = lax.shift_right_logical(cnt + (LANES - 1), 4)

    def extract_band(blg, par, fcount):
        parv = jnp.full((LANES,), 1, jnp.int32) * par

        def chunk(c, fc):
            hc = hl[pl.ds(c * LANES, LANES)]
            m = lax.shift_right_logical(hc, 22) == (
                jnp.full((LANES,), 1, jnp.int32) * blg[0])
            col = lax.shift_right_logical(hc, 15) & c127

            def do_extract(fc2):
                slot = lax.rem(fc2, 8)
                arow = slot * LANES + lanes
                colv = col + blg[1]

                @pl.when(fc2 >= 8)
                def _():
                    pltpu.make_async_copy(
                        g_hbm.at[pl.ds(0, LANES), :],
                        arena.at[pl.ds(0, LANES), :], ssem).wait()

                for f in range(DIM):
                    fv = jnp.full((LANES,), f, jnp.int32)
                    vals = plsc.load_gather(rb, [parv, fv, colv])
                    plsc.store_scatter(arena, [arow, fv], vals)
                psel = jnp.where(m, hc & jnp.full((LANES,), 0x7FFF,
                                                  jnp.int32),
                                 jnp.full((LANES,), SENT, jnp.int32))

                pltpu.async_copy(
                    arena.at[pl.ds(slot * LANES, LANES), :],
                    g_hbm.at[psel], ssem)
                return fc2 + 1

            return lax.cond(jnp.any(m), do_extract, lambda x: x, fc)

        return lax.fori_loop(0, nch, chunk, fcount)

    def fetch(s, par):
        start = jnp.minimum(lo_w + s * SUPER, NFULL - SUPER)
        return pltpu.async_copy(
            table_hbm.at[:, pl.ds(start * BAND, SUPER * BAND)],
            rb.at[par], fsem)

    fetch(jnp.int32(0), jnp.int32(0))

    def super_step(s, fcount):
        par = lax.rem(s, 2)
        pltpu.make_async_copy(
            table_hbm.at[:, pl.ds(0, SUPER * BAND)], rb.at[0], fsem).wait()

        @pl.when(s + 1 < NSUPER)
        def _():
            fetch(s + 1, lax.rem(s + 1, 2))

        start = jnp.minimum(lo_w + s * SUPER, NFULL - SUPER)
        for sl in range(SUPER):
            fcount = extract_band((start + sl - lo_w, sl * BAND), par,
                                  fcount)
        return fcount

    fcount = lax.fori_loop(0, NSUPER, super_step, jnp.int32(0))

    @pl.when(wid == NW - 1)
    def _():
        pltpu.sync_copy(edge_hbm, rb.at[0, :, pl.ds(0, BAND)])

    fcount = lax.cond(
        wid == NW - 1,
        lambda fc: extract_band((NFULL - lo_w, 0), jnp.int32(0), fc),
        lambda fc: fc,
        fcount)

    def drain(_, carry):
        pltpu.make_async_copy(
            g_hbm.at[pl.ds(0, LANES), :],
            arena.at[pl.ds(0, LANES), :], ssem).wait()
        return carry

    lax.fori_loop(0, jnp.minimum(fcount, 8), drain, 0)


def _body_a(u_hbm, i_hbm, j_hbm, ufT, ifT, ufE, ifE,
            gu_hbm, gi_hbm, gj_hbm, xbuf, hl, rb, arena, fsem, ssem):
    wid = lax.axis_index("s") * NUM_CORES + lax.axis_index("c")
    lo_w = wid * 244 + jnp.minimum(wid, 5)
    nb_w = jnp.where(wid < 5, 245, 244)
    _scan_pass(u_hbm, ufT, ufE, gu_hbm, xbuf, hl, rb, arena, fsem, ssem,
               wid, lo_w, nb_w)
    _scan_pass(i_hbm, ifT, ifE, gi_hbm, xbuf, hl, rb, arena, fsem, ssem,
               wid, lo_w, nb_w)
    _scan_pass(j_hbm, ifT, ifE, gj_hbm, xbuf, hl, rb, arena, fsem, ssem,
               wid, lo_w, nb_w)


def _body_b(i_hbm, j_hbm, gu_hbm, gi_hbm, gj_hbm, bias_hbm, out_hbm,
            ii, ji, bu, bv, bw, bi, bj, ov, sem):
    wid = lax.axis_index("s") * NUM_CORES + lax.axis_index("c")
    base = wid * RPW
    pltpu.sync_copy(i_hbm.at[pl.ds(base, RPW)], ii)
    pltpu.sync_copy(j_hbm.at[pl.ds(base, RPW)], ji)
    c4 = pltpu.async_copy(bias_hbm.at[ii], bi, sem)
    c5 = pltpu.async_copy(bias_hbm.at[ji], bj, sem)
    c4.wait()
    c5.wait()

    lanes = lax.iota(jnp.int32, LANES)
    HALF = RPW // 2

    for h in range(2):
        hb = base + h * HALF
        pltpu.sync_copy(gu_hbm.at[pl.ds(hb, HALF), :], bu)
        pltpu.sync_copy(gi_hbm.at[pl.ds(hb, HALF), :], bv)
        pltpu.sync_copy(gj_hbm.at[pl.ds(hb, HALF), :], bw)

        def group(g, carry):
            rb_ = g * LANES
            row_idx = lanes + rb_
            ob = h * HALF + rb_
            acc = bi[pl.ds(ob, LANES)] - bj[pl.ds(ob, LANES)]

            def dstep(d, a):
                dv = jnp.full((LANES,), d, jnp.int32)
                uu = plsc.load_gather(bu, [row_idx, dv])
                xi = plsc.load_gather(bv, [row_idx, dv])
                xj = plsc.load_gather(bw, [row_idx, dv])
                return a + uu * (xi - xj)

            acc = lax.fori_loop(0, DIM, dstep, acc, unroll=8)
            ov[pl.ds(ob, LANES)] = acc
            return carry

        lax.fori_loop(0, HALF // LANES, group, 0)

    pltpu.sync_copy(ov, out_hbm.at[pl.ds(base, RPW)])


@functools.partial(jax.jit, static_argnames=())
def kernel(u, i, j, user_factors, item_factors, item_biases):
    mesh = plsc.VectorSubcoreMesh(core_axis_name="c", subcore_axis_name="s")
    cp = pltpu.CompilerParams(needs_layout_passes=False)

    ka = functools.partial(
        pl.kernel,
        mesh=mesh,
        compiler_params=cp,
        out_type=(
            jax.ShapeDtypeStruct((GROWS, PW), jnp.float32),
            jax.ShapeDtypeStruct((GROWS, PW), jnp.float32),
            jax.ShapeDtypeStruct((GROWS, PW), jnp.float32),
        ),
        scratch_types=[
            pltpu.VMEM((B,), jnp.int32),
            pltpu.VMEM((B + LANES,), jnp.int32),
            pltpu.VMEM((2, DIM, SUPER * BAND), jnp.float32),
            pltpu.VMEM((8 * LANES, PW), jnp.float32),
            pltpu.SemaphoreType.DMA,
            pltpu.SemaphoreType.DMA,
        ],
    )(_body_a)

    kb = functools.partial(
        pl.kernel,
        mesh=mesh,
        compiler_params=cp,
        out_type=jax.ShapeDtypeStruct((B,), jnp.float32),
        scratch_types=[
            pltpu.VMEM((RPW,), jnp.int32),
            pltpu.VMEM((RPW,), jnp.int32),
            pltpu.VMEM((RPW // 2, PW), jnp.float32),
            pltpu.VMEM((RPW // 2, PW), jnp.float32),
            pltpu.VMEM((RPW // 2, PW), jnp.float32),
            pltpu.VMEM((RPW,), jnp.float32),
            pltpu.VMEM((RPW,), jnp.float32),
            pltpu.VMEM((RPW,), jnp.float32),
            pltpu.SemaphoreType.DMA,
        ],
    )(_body_b)

    ufT = user_factors.T
    ifT = item_factors.T
    ufE = jnp.pad(user_factors[EDGE0:].T, ((0, 0), (0, BAND - DIM)))
    ifE = jnp.pad(item_factors[EDGE0:].T, ((0, 0), (0, BAND - DIM)))
    bias_flat = item_biases.reshape(-1)
    gu, gi, gj = ka(u, i, j, ufT, ifT, ufE, ifE)
    return kb(i, j, gu, gi, gj, bias_flat)
